# baseline (speedup 1.0000x reference)
import jax
import jax.numpy as jnp
from jax.experimental import pallas as pl

_BLOCK_ROWS = 15000


def _copy_body(u_ref, i_ref, ou_ref, oi_ref):
    ou_ref[...] = u_ref[...]
    oi_ref[...] = i_ref[...]


def kernel(embed_user, embed_item):
    n_u, e = embed_user.shape
    n_i, _ = embed_item.shape
    assert n_u == n_i, "single-grid copy assumes equal table heights"
    grid = (-(-n_u // _BLOCK_ROWS),)
    g = grid[0]
    spec = pl.BlockSpec((_BLOCK_ROWS, e), lambda i: (i, 0))
    rspec = pl.BlockSpec((_BLOCK_ROWS, e), lambda i: (g - 1 - i, 0))
    out_u, out_i = pl.pallas_call(
        _copy_body,
        grid=grid,
        in_specs=[spec, rspec],
        out_specs=[spec, rspec],
        out_shape=[
            jax.ShapeDtypeStruct((n_u, e), embed_user.dtype),
            jax.ShapeDtypeStruct((n_i, e), embed_item.dtype),
        ],
    )(embed_user, embed_item)
    return (out_u, out_i)


# speedup vs baseline: 1.0109x; 1.0109x over previous
import jax
import jax.numpy as jnp
from jax.experimental import pallas as pl

_BLOCK_ROWS = 15000


def _copy_body(u_ref, i_ref, ou_ref, oi_ref):
    ou_ref[...] = u_ref[...]
    oi_ref[...] = i_ref[...]


def kernel(embed_user, embed_item):
    n_u, e = embed_user.shape
    n_i, _ = embed_item.shape
    assert n_u == n_i, "single-grid copy assumes equal table heights"
    grid = (-(-n_u // _BLOCK_ROWS),)
    spec = pl.BlockSpec((_BLOCK_ROWS, e), lambda i: (i, 0))
    out_u, out_i = pl.pallas_call(
        _copy_body,
        grid=grid,
        in_specs=[spec, spec],
        out_specs=[spec, spec],
        out_shape=[
            jax.ShapeDtypeStruct((n_u, e), embed_user.dtype),
            jax.ShapeDtypeStruct((n_i, e), embed_item.dtype),
        ],
    )(embed_user, embed_item)
    return (out_u, out_i)
